# baseline (speedup 1.0000x reference)
import jax
import jax.numpy as jnp
from jax import lax
from jax.experimental import pallas as pl
from jax.experimental.pallas import tpu as pltpu
from jax.experimental.pallas import tpu_sc as plsc

_N = 1000000
_H = 16
_C = 6
_B = 16384

_NC = 2
_NS = 16
_NW = _NC * _NS
_BPW = _B // _NW


def _body(ue_hbm, ve_hbm, w_hbm, u_tab, v_tab, out_hbm,
          ue_v, ve_v, u_cols, v_cols, w_v, out_t, sem_u, sem_v):
  wid = lax.axis_index("s") * _NC + lax.axis_index("c")
  base = wid * _BPW
  pltpu.sync_copy(ue_hbm.at[wid], ue_v)
  pltpu.sync_copy(ve_hbm.at[wid], ve_v)
  pltpu.sync_copy(w_hbm, w_v)

  cu = pltpu.async_copy(u_tab.at[ue_v], u_cols, sem_u)
  cv = pltpu.async_copy(v_tab.at[ve_v], v_cols, sem_v)
  cu.wait()
  cv.wait()

  def blk_body(blk, _):
    j0 = blk * 16
    acc = [w_v[2 * _H * _C + c, :] for c in range(_C)]
    for h in range(_H):
      cu16 = u_cols[pl.ds(h * _BPW + j0, 16)]
      cv16 = v_cols[pl.ds(h * _BPW + j0, 16)]
      for c in range(_C):
        acc[c] = acc[c] + cu16 * w_v[h * _C + c, :]
        acc[c] = acc[c] + cv16 * w_v[(_H + h) * _C + c, :]
    for c in range(_C):
      out_t[c, pl.ds(j0, 16)] = acc[c]
    return 0

  lax.fori_loop(0, _BPW // 16, blk_body, 0)

  pltpu.sync_copy(out_t, out_hbm.at[:, pl.ds(base, _BPW)])


_sc_fused = pl.kernel(
    _body,
    out_type=jax.ShapeDtypeStruct((_C, _B), jnp.float32),
    mesh=plsc.VectorSubcoreMesh(core_axis_name="c", subcore_axis_name="s"),
    scratch_types=[
        pltpu.VMEM((_H * _BPW,), jnp.int32),
        pltpu.VMEM((_H * _BPW,), jnp.int32),
        pltpu.VMEM((_H * _BPW,), jnp.float32),
        pltpu.VMEM((_H * _BPW,), jnp.float32),
        pltpu.VMEM((2 * _H * _C + _C, 16), jnp.float32),
        pltpu.VMEM((_C, _BPW), jnp.float32),
        pltpu.SemaphoreType.DMA,
        pltpu.SemaphoreType.DMA,
    ],
)


@jax.jit
def kernel(X_batch, U, V, W, b):
  x0 = X_batch[:, 0].astype(jnp.int32)
  x1 = X_batch[:, 1].astype(jnp.int32)
  h_iota = jnp.arange(_H, dtype=jnp.int32)[None, :, None]
  ue = (x0.reshape(_NW, 1, _BPW) * _H + h_iota).reshape(_NW, _H * _BPW)
  ve = (x1.reshape(_NW, 1, _BPW) * _H + h_iota).reshape(_NW, _H * _BPW)
  u_tab = U.reshape(_N * _H)
  v_tab = V.reshape(_N * _H)
  wb = jnp.broadcast_to(jnp.concatenate([W.reshape(-1), b])[:, None],
                        (2 * _H * _C + _C, 16))
  out_t = _sc_fused(ue, ve, wb, u_tab, v_tab)
  return out_t.T


# speedup vs baseline: 1.4035x; 1.4035x over previous
import jax
import jax.numpy as jnp
from jax import lax
from jax.experimental import pallas as pl
from jax.experimental.pallas import tpu as pltpu
from jax.experimental.pallas import tpu_sc as plsc

_N = 1000000
_H = 16
_C = 6
_B = 16384

_NC = 2
_NS = 16
_NW = _NC * _NS
_BPW = _B // _NW
_PH = _BPW // 2


def _body(ur_hbm, vr_hbm, w_hbm, u_tab, v_tab, out_hbm,
          ur_v, vr_v, rows_u, rows_v, w_v, out_t, sem_u, sem_v):
  wid = lax.axis_index("s") * _NC + lax.axis_index("c")
  base = wid * _BPW
  pltpu.sync_copy(ur_hbm.at[wid], ur_v)
  pltpu.sync_copy(vr_hbm.at[wid], vr_v)
  pltpu.sync_copy(w_hbm, w_v)

  iota = lax.broadcasted_iota(jnp.int32, (16,), 0)

  for phase in range(2):
    p0 = phase * _PH

    def fetch_body(blk, _, p0=p0):
      j0 = blk * 16
      ru_vec = ur_v[pl.ds(p0 + j0, 16)]
      rv_vec = vr_v[pl.ds(p0 + j0, 16)]
      copies = []
      for k in range(16):
        ru_k = jnp.sum(jnp.where(iota == k, ru_vec, 0))
        copies.append(pltpu.async_copy(u_tab.at[ru_k], rows_u.at[j0 + k],
                                       sem_u))
      for c in copies:
        c.wait()
      copies = []
      for k in range(16):
        rv_k = jnp.sum(jnp.where(iota == k, rv_vec, 0))
        copies.append(pltpu.async_copy(v_tab.at[rv_k], rows_v.at[j0 + k],
                                       sem_v))
      for c in copies:
        c.wait()
      return 0

    lax.fori_loop(0, _PH // 16, fetch_body, 0)

    def blk_body(blk, _, p0=p0):
      j0 = blk * 16
      rowv = j0 + iota
      acc = [w_v[2 * _H * _C + c, :] for c in range(_C)]
      for h in range(_H):
        hv = jnp.full((16,), h, dtype=jnp.int32)
        cu16 = plsc.load_gather(rows_u, [rowv, hv])
        cv16 = plsc.load_gather(rows_v, [rowv, hv])
        for c in range(_C):
          acc[c] = acc[c] + cu16 * w_v[h * _C + c, :]
          acc[c] = acc[c] + cv16 * w_v[(_H + h) * _C + c, :]
      for c in range(_C):
        out_t[c, pl.ds(p0 + j0, 16)] = acc[c]
      return 0

    lax.fori_loop(0, _PH // 16, blk_body, 0)

  pltpu.sync_copy(out_t, out_hbm.at[:, pl.ds(base, _BPW)])


_sc_fused = pl.kernel(
    _body,
    out_type=jax.ShapeDtypeStruct((_C, _B), jnp.float32),
    mesh=plsc.VectorSubcoreMesh(core_axis_name="c", subcore_axis_name="s"),
    compiler_params=pltpu.CompilerParams(needs_layout_passes=False),
    scratch_types=[
        pltpu.VMEM((_BPW,), jnp.int32),
        pltpu.VMEM((_BPW,), jnp.int32),
        pltpu.VMEM((_PH, _H), jnp.float32),
        pltpu.VMEM((_PH, _H), jnp.float32),
        pltpu.VMEM((2 * _H * _C + _C, 16), jnp.float32),
        pltpu.VMEM((_C, _BPW), jnp.float32),
        pltpu.SemaphoreType.DMA,
        pltpu.SemaphoreType.DMA,
    ],
)


@jax.jit
def kernel(X_batch, U, V, W, b):
  x0 = X_batch[:, 0].astype(jnp.int32)
  x1 = X_batch[:, 1].astype(jnp.int32)
  ur = x0.reshape(_NW, _BPW)
  vr = x1.reshape(_NW, _BPW)
  wb = jnp.broadcast_to(jnp.concatenate([W.reshape(-1), b])[:, None],
                        (2 * _H * _C + _C, 16))
  out_t = _sc_fused(ur, vr, wb, U, V)
  return out_t.T
